# baseline (speedup 1.0000x reference)
import functools

import jax
import jax.numpy as jnp
from jax import lax
from jax.experimental import pallas as pl
from jax.experimental.pallas import tpu as pltpu
from jax.experimental.pallas import tpu_sc as plsc

D = 128
L = 200
PCOLS = 16
PACK = D // PCOLS
NC, NS = 2, 16
NW = NC * NS


def _project_body(t_ref, w_ref, b_ref, o_ref):
    acc = b_ref[...] + jnp.dot(t_ref[0], w_ref[0],
                               preferred_element_type=jnp.float32)
    for j in range(1, PACK):
        acc = acc + jnp.dot(t_ref[j], w_ref[j],
                            preferred_element_type=jnp.float32)
    o_ref[...] = acc


def _project(table_g, wbd, brow):
    tm = 5000
    vg = table_g.shape[1]
    return pl.pallas_call(
        _project_body,
        grid=(vg // tm,),
        in_specs=[
            pl.BlockSpec((PACK, tm, D), lambda i: (0, i, 0)),
            pl.BlockSpec((PACK, D, D), lambda i: (0, 0, 0)),
            pl.BlockSpec((1, D), lambda i: (0, 0)),
        ],
        out_specs=pl.BlockSpec((tm, D), lambda i: (i, 0)),
        out_shape=jax.ShapeDtypeStruct((vg, D), jnp.float32),
        compiler_params=pltpu.CompilerParams(
            dimension_semantics=("arbitrary",)),
    )(table_g, wbd, brow)


NL = 4
NBLK = L // NL
Q = 4


def _make_pool(batch):
    rows_per_w = batch // NW
    o_per_w = rows_per_w * PCOLS // 128

    @functools.partial(
        pl.kernel,
        out_type=jax.ShapeDtypeStruct((batch * PCOLS // 128, 128),
                                      jnp.float32),
        mesh=plsc.VectorSubcoreMesh(core_axis_name="c", subcore_axis_name="s",
                                    num_cores=NC, num_subcores=NS),
        scratch_types=[
            pltpu.VMEM((2, Q, 2 * NL, 128), jnp.int32),
            pltpu.VMEM((2, NL * rows_per_w, PCOLS), jnp.float32),
            pltpu.VMEM((rows_per_w, PCOLS), jnp.float32),
            pltpu.VMEM((64, 128), jnp.float32),
            pltpu.SemaphoreType.DMA,
            pltpu.SemaphoreType.DMA,
            pltpu.SemaphoreType.DMA,
            pltpu.SemaphoreType.DMA,
        ],
        compiler_params=pltpu.CompilerParams(use_tc_tiling_on_sc=False),
    )
    def pool(text_hbm, proj_hbm, out_hbm,
             idx_v, rows_v, acc_v, ostage_v,
             gsem0, gsem1, isem0, isem1):
        gsems = (gsem0, gsem1)
        isems = (isem0, isem1)
        sid = lax.axis_index("s")
        wid = sid * NC + lax.axis_index("c")
        obase = wid * o_per_w

        cb4 = wid * Q

        def issue_idx(f, slot):
            for q in range(Q):
                pltpu.async_copy(text_hbm.at[f, cb4 + q],
                                 idx_v.at[slot, q], isems[slot])

        def wait_idx(f, slot):
            for q in range(Q):
                pltpu.make_async_copy(text_hbm.at[f, cb4 + q],
                                      idx_v.at[slot, q], isems[slot]).wait()

        def issue_gathers(slot, fslot, loff):
            for q in range(Q):
                for li in range(NL):
                    pltpu.async_copy(
                        proj_hbm.at[idx_v.at[fslot, q, loff + li]],
                        rows_v.at[slot, pl.ds((li * Q + q) * 128, 128)],
                        gsems[slot])

        def wait_gathers(slot, fslot, loff):
            for q in range(Q):
                for li in range(NL):
                    pltpu.make_async_copy(
                        proj_hbm.at[idx_v.at[fslot, q, loff + li]],
                        rows_v.at[slot, pl.ds((li * Q + q) * 128, 128)],
                        gsems[slot]).wait()

        def accumulate(slot):
            def jbody(j, carry):
                for q in range(Q):
                    base = q * 128 + j
                    v = rows_v[slot, base]
                    for li in range(1, NL):
                        v = v + rows_v[slot, li * Q * 128 + base]
                    plsc.addupdate(acc_v.at[base], v)
                return carry

            lax.fori_loop(0, 128, jbody, 0)

        def zrow(i, carry):
            acc_v[i] = jnp.zeros((PCOLS,), jnp.float32)
            return carry

        lax.fori_loop(0, rows_per_w, zrow, 0)

        issue_idx(0, 0)
        wait_idx(0, 0)
        issue_gathers(0, 0, 0)

        def body(k, carry):
            f2 = 2 * k
            wait_gathers(0, 0, 0)
            issue_idx(f2 + 1, 1)
            issue_gathers(1, 0, NL)
            accumulate(0)
            wait_gathers(1, 0, NL)
            wait_idx(f2 + 1, 1)
            issue_gathers(0, 1, 0)
            accumulate(1)
            wait_gathers(0, 1, 0)
            issue_idx(f2 + 2, 0)
            issue_gathers(1, 1, NL)
            accumulate(0)
            wait_gathers(1, 1, NL)
            wait_idx(f2 + 2, 0)
            issue_gathers(0, 0, 0)
            accumulate(1)
            return carry

        lax.fori_loop(0, (NBLK - 2) // 4, body, 0)

        wait_gathers(0, 0, 0)
        issue_gathers(1, 0, NL)
        accumulate(0)
        wait_gathers(1, 0, NL)
        accumulate(1)

        def orow(g, carry):
            for j in range(8):
                ostage_v[g, pl.ds(j * PCOLS, PCOLS)] = acc_v[g * 8 + j]
            return carry

        lax.fori_loop(0, o_per_w, orow, 0)
        pltpu.sync_copy(ostage_v, out_hbm.at[pl.ds(obase, o_per_w)])

    return pool


def kernel(text, emb_table, fc_w, fc_b):
    batch = text.shape[0]
    vocab = emb_table.shape[0]
    vg = vocab // PACK
    ncls = fc_w.shape[1]
    t32 = text.astype(jnp.int32)
    r32 = (t32 % vg) * PACK + t32 // vg
    text4 = r32.reshape(batch // 128, 128, L // 8, 8).transpose(2, 0, 3, 1)
    w16 = jnp.pad(fc_w, ((0, 0), (0, PCOLS - ncls))) * (1.0 / L)
    wbd = jnp.kron(jnp.eye(PACK, dtype=jnp.float32), w16).reshape(PACK, D, D)
    brow = jnp.tile(jnp.pad(fc_b, (0, PCOLS - ncls)) * (1.0 / L),
                    PACK)[None, :]
    table_g = emb_table.reshape(PACK, vg, D)
    proj = _project(table_g, wbd, brow).reshape(vocab, PCOLS)
    out = _make_pool(batch)(text4, proj)
    return out.reshape(batch, PCOLS)[:, :ncls]
